# Initial kernel scaffold; baseline (speedup 1.0000x reference)
import jax
import jax.numpy as jnp
from jax.experimental import pallas as pl


def kernel(x, L, W1, b1, W2, b2, thetas, W3, b3):
    raise NotImplementedError("write your pallas kernel here")


# speedup vs baseline: 3.0411x; 3.0411x over previous
from functools import partial
from math import comb

import numpy as np

import jax
import jax.numpy as jnp
from jax.experimental import pallas as pl

_LANE = 128


def _monomial_matrix(k: int) -> np.ndarray:
    m = np.zeros((k + 1, k + 1), dtype=np.float64)
    for j in range(k + 1):
        base = comb(k, j) / (2.0 ** k)
        for t in range(k - j + 1):
            m[j + t, j] += base * comb(k - j, t) * (2.0 ** (k - j - t)) * ((-1.0) ** t)
    return m


def _pick_rows(n: int, target: int) -> int:
    for mult in (16, 8, 1):
        for b in range(min(target, n), 0, -1):
            if n % b == 0 and b % mult == 0:
                return b
    return n


def _mlp_body(x_ref, w1_ref, b1_ref, w2_ref, b2_ref, o_ref):
    h = jnp.dot(x_ref[...], w1_ref[...], preferred_element_type=jnp.float32)
    h = jnp.maximum(h + b1_ref[...], 0.0)
    o_ref[...] = jnp.dot(h, w2_ref[...], preferred_element_type=jnp.float32) + b2_ref[...]


def _first_prop_body(l_ref, ub_ref, ui_ref, acc_ref, c_ref,
                     uo_ref, acco_ref, e_ref, *, bm, relu_out):
    i = pl.program_id(0)
    rows = i * bm + jax.lax.broadcasted_iota(jnp.int32, l_ref.shape, 0)
    cols = jax.lax.broadcasted_iota(jnp.int32, l_ref.shape, 1)
    e = (l_ref[...] - (rows == cols).astype(jnp.float32)).astype(jnp.bfloat16)
    e_ref[...] = e
    new_u = ui_ref[...] + jnp.dot(e, ub_ref[...], preferred_element_type=jnp.float32)
    uo_ref[...] = new_u
    a = acc_ref[...] + c_ref[...] * new_u
    acco_ref[...] = jnp.maximum(a, 0.0) if relu_out else a


def _prop_body(e_ref, ub_ref, ui_ref, acc_ref, c_ref, uo_ref, acco_ref, *, relu_out):
    new_u = ui_ref[...] + jnp.dot(e_ref[...], ub_ref[...],
                                  preferred_element_type=jnp.float32)
    uo_ref[...] = new_u
    a = acc_ref[...] + c_ref[...] * new_u
    acco_ref[...] = jnp.maximum(a, 0.0) if relu_out else a


def _out_body(h_ref, w_ref, b_ref, o_ref):
    o_ref[...] = jnp.dot(h_ref[...], w_ref[...],
                         preferred_element_type=jnp.float32) + b_ref[...]


def _first_prop(L, u_bf, u, acc, c_tile, relu_out):
    n, f = u.shape
    bm = _pick_rows(n, 400)
    return pl.pallas_call(
        partial(_first_prop_body, bm=bm, relu_out=relu_out),
        grid=(n // bm,),
        in_specs=[
            pl.BlockSpec((bm, n), lambda i: (i, 0)),
            pl.BlockSpec((n, f), lambda i: (0, 0)),
            pl.BlockSpec((bm, f), lambda i: (i, 0)),
            pl.BlockSpec((bm, f), lambda i: (i, 0)),
            pl.BlockSpec((1, f), lambda i: (0, 0)),
        ],
        out_specs=[
            pl.BlockSpec((bm, f), lambda i: (i, 0)),
            pl.BlockSpec((bm, f), lambda i: (i, 0)),
            pl.BlockSpec((bm, n), lambda i: (i, 0)),
        ],
        out_shape=[
            jax.ShapeDtypeStruct((n, f), jnp.float32),
            jax.ShapeDtypeStruct((n, f), jnp.float32),
            jax.ShapeDtypeStruct((n, n), jnp.bfloat16),
        ],
    )(L, u_bf, u, acc, c_tile)


def _prop(E, u_bf, u, acc, c_tile, relu_out):
    n, f = u.shape
    bm = _pick_rows(n, 400)
    return pl.pallas_call(
        partial(_prop_body, relu_out=relu_out),
        grid=(n // bm,),
        in_specs=[
            pl.BlockSpec((bm, n), lambda i: (i, 0)),
            pl.BlockSpec((n, f), lambda i: (0, 0)),
            pl.BlockSpec((bm, f), lambda i: (i, 0)),
            pl.BlockSpec((bm, f), lambda i: (i, 0)),
            pl.BlockSpec((1, f), lambda i: (0, 0)),
        ],
        out_specs=[
            pl.BlockSpec((bm, f), lambda i: (i, 0)),
            pl.BlockSpec((bm, f), lambda i: (i, 0)),
        ],
        out_shape=[
            jax.ShapeDtypeStruct((n, f), jnp.float32),
            jax.ShapeDtypeStruct((n, f), jnp.float32),
        ],
    )(E, u_bf, u, acc, c_tile)


def kernel(x, L, W1, b1, W2, b2, thetas, W3, b3):
    n, fin = x.shape
    hdim = W2.shape[1]
    k_order = thetas.shape[1] - 1
    num_layers = thetas.shape[0]

    mono = jnp.asarray(_monomial_matrix(k_order), dtype=jnp.float32)
    coeffs = (mono @ thetas.T).T

    bm0 = _pick_rows(n, 1000)
    h = pl.pallas_call(
        _mlp_body,
        grid=(n // bm0,),
        in_specs=[
            pl.BlockSpec((bm0, fin), lambda i: (i, 0)),
            pl.BlockSpec(W1.shape, lambda i: (0, 0)),
            pl.BlockSpec((1, hdim), lambda i: (0, 0)),
            pl.BlockSpec(W2.shape, lambda i: (0, 0)),
            pl.BlockSpec((1, hdim), lambda i: (0, 0)),
        ],
        out_specs=pl.BlockSpec((bm0, hdim), lambda i: (i, 0)),
        out_shape=jax.ShapeDtypeStruct((n, hdim), jnp.float32),
    )(x, W1, b1.reshape(1, -1), W2, b2.reshape(1, -1))

    e_mat = None
    for l in range(num_layers):
        acc = coeffs[l, 0] * h
        u = h
        for m in range(1, k_order + 1):
            u_bf = u.astype(jnp.bfloat16)
            c_tile = jnp.full((1, hdim), coeffs[l, m], dtype=jnp.float32)
            relu_out = m == k_order
            if e_mat is None:
                u, acc, e_mat = _first_prop(L, u_bf, u, acc, c_tile, relu_out)
            else:
                u, acc = _prop(e_mat, u_bf, u, acc, c_tile, relu_out)
        h = acc

    c_out = W3.shape[1]
    pad = (-c_out) % _LANE
    W3p = jnp.pad(W3, ((0, 0), (0, pad)))
    b3p = jnp.pad(b3, (0, pad)).reshape(1, -1)
    y = pl.pallas_call(
        _out_body,
        grid=(n // bm0,),
        in_specs=[
            pl.BlockSpec((bm0, hdim), lambda i: (i, 0)),
            pl.BlockSpec(W3p.shape, lambda i: (0, 0)),
            pl.BlockSpec((1, c_out + pad), lambda i: (0, 0)),
        ],
        out_specs=pl.BlockSpec((bm0, c_out + pad), lambda i: (i, 0)),
        out_shape=jax.ShapeDtypeStruct((n, c_out + pad), jnp.float32),
    )(h, W3p, b3p)
    return y[:, :c_out] if pad else y
